# Initial kernel scaffold; baseline (speedup 1.0000x reference)
import jax
import jax.numpy as jnp
from jax.experimental import pallas as pl


def kernel(input_ids, token_table, pos_table, ln_gamma, ln_beta):
    raise NotImplementedError("write your pallas kernel here")


# speedup vs baseline: 1.4073x; 1.4073x over previous
import functools

import jax
import jax.numpy as jnp
from jax import lax
from jax.experimental import pallas as pl
from jax.experimental.pallas import tpu as pltpu
from jax.experimental.pallas import tpu_sc as plsc

VOCAB = 100000
HIDDEN = 128
MAX_POS = 512
BATCH = 1024
SEQ = 200

L = 16
NW = 32
ROWS_PER_BLOCK = 100
BLOCKS = (BATCH * SEQ) // ROWS_PER_BLOCK
BLOCKS_PER_W = BLOCKS // NW
GROUPS = (ROWS_PER_BLOCK + L - 1) // L
BUF_ROWS = GROUPS * L
POS_BUF = SEQ + L
KV = HIDDEN // L


def _rsqrt(x):
    xi = lax.bitcast_convert_type(x, jnp.int32)
    yi = jnp.int32(0x5F3759DF) - (xi >> 1)
    y = lax.bitcast_convert_type(yi, jnp.float32)
    for _ in range(3):
        y = y * (1.5 - 0.5 * x * y * y)
    return y


def _sc_embed_ln(ids2, token_table, pos_table, ln_gamma, ln_beta):
    mesh = plsc.VectorSubcoreMesh(core_axis_name="c", subcore_axis_name="s")

    @functools.partial(
        pl.kernel,
        mesh=mesh,
        out_type=jax.ShapeDtypeStruct((BLOCKS, ROWS_PER_BLOCK, HIDDEN),
                                      jnp.float32),
        compiler_params=pltpu.CompilerParams(needs_layout_passes=False),
        scratch_types=[
            pltpu.VMEM((BLOCKS_PER_W, ROWS_PER_BLOCK), jnp.int32),
            pltpu.VMEM((BUF_ROWS, HIDDEN), jnp.float32),
            pltpu.VMEM((POS_BUF, HIDDEN), jnp.float32),
            pltpu.VMEM((HIDDEN,), jnp.float32),
            pltpu.VMEM((HIDDEN,), jnp.float32),
            pltpu.VMEM((L * L,), jnp.float32),
            pltpu.VMEM((L * L,), jnp.float32),
            pltpu.VMEM((L,), jnp.float32),
            pltpu.VMEM((L,), jnp.float32),
            pltpu.SemaphoreType.DMA,
        ],
    )
    def k(ids_hbm, table_hbm, pos_hbm, gamma_hbm, beta_hbm, out_hbm,
          idx_v, buf, pos_v, gamma_v, beta_v, smat_s, smat_q,
          stage_a, stage_b, sem):
        wid = lax.axis_index("s") * 2 + lax.axis_index("c")

        pltpu.sync_copy(ids_hbm.at[pl.ds(wid * BLOCKS_PER_W, BLOCKS_PER_W)],
                        idx_v)
        pltpu.sync_copy(pos_hbm.at[pl.ds(0, SEQ)], pos_v.at[pl.ds(0, SEQ)])
        pltpu.sync_copy(gamma_hbm, gamma_v)
        pltpu.sync_copy(beta_hbm, beta_v)

        def block_body(u, _):
            blk = wid * BLOCKS_PER_W + u
            p0 = (u % 2) * ROWS_PER_BLOCK

            pltpu.async_copy(table_hbm.at[idx_v.at[u]],
                             buf.at[pl.ds(0, ROWS_PER_BLOCK)], sem).wait()

            def group_body(g, _):
                r0 = g * L
                for r in range(L):
                    row = r0 + r
                    acc_s = None
                    acc_q = None
                    for kk in range(KV):
                        sl = pl.ds(kk * L, L)
                        x = buf[row, sl] + pos_v[p0 + row, sl]
                        buf[row, sl] = x
                        acc_s = x if acc_s is None else acc_s + x
                        acc_q = x * x if acc_q is None else acc_q + x * x
                    smat_s[pl.ds(r * L, L)] = acc_s
                    smat_q[pl.ds(r * L, L)] = acc_q
                col_base = lax.iota(jnp.int32, L) * L
                sv = None
                qv = None
                for j in range(L):
                    cidx = col_base + j
                    cs = plsc.load_gather(smat_s, [cidx])
                    cq = plsc.load_gather(smat_q, [cidx])
                    sv = cs if sv is None else sv + cs
                    qv = cq if qv is None else qv + cq
                mean = sv * (1.0 / HIDDEN)
                var = qv * (1.0 / HIDDEN) - mean * mean
                rstd = _rsqrt(var + 1e-12)
                stage_a[...] = rstd
                stage_b[...] = -mean * rstd
                gv = [gamma_v[pl.ds(kk * L, L)] for kk in range(KV)]
                bv = [beta_v[pl.ds(kk * L, L)] for kk in range(KV)]
                av_all = stage_a[...]
                bv_all = stage_b[...]
                for r in range(L):
                    row = r0 + r
                    a = av_all[r]
                    b = bv_all[r]
                    for kk in range(KV):
                        sl = pl.ds(kk * L, L)
                        x = buf[row, sl]
                        buf[row, sl] = (x * a + b) * gv[kk] + bv[kk]
                return 0

            lax.fori_loop(0, GROUPS, group_body, 0)

            pltpu.sync_copy(buf.at[pl.ds(0, ROWS_PER_BLOCK)], out_hbm.at[blk])
            return 0

        lax.fori_loop(0, BLOCKS_PER_W, block_body, 0)

    return k(ids2, token_table, pos_table, ln_gamma, ln_beta)


def kernel(input_ids, token_table, pos_table, ln_gamma, ln_beta):
    ids2 = input_ids.astype(jnp.int32).reshape(BLOCKS, ROWS_PER_BLOCK)
    out = _sc_embed_ln(ids2, token_table, pos_table, ln_gamma, ln_beta)
    return out.reshape(BATCH, SEQ, HIDDEN)


# speedup vs baseline: 2.7463x; 1.9515x over previous
import functools

import jax
import jax.numpy as jnp
from jax import lax
from jax.experimental import pallas as pl
from jax.experimental.pallas import tpu as pltpu
from jax.experimental.pallas import tpu_sc as plsc

VOCAB = 100000
HIDDEN = 128
MAX_POS = 512
BATCH = 1024
SEQ = 200

L = 16
NW = 32
ROWS_PER_BLOCK = 100
BLOCKS = (BATCH * SEQ) // ROWS_PER_BLOCK
BLOCKS_PER_W = BLOCKS // NW
GROUPS = (ROWS_PER_BLOCK + L - 1) // L
BUF_ROWS = GROUPS * L
POS_BUF = SEQ + L
KV = HIDDEN // L
NBUF = 2
SUPERS = BLOCKS_PER_W // NBUF


def _rsqrt(x):
    xi = lax.bitcast_convert_type(x, jnp.int32)
    yi = jnp.int32(0x5F3759DF) - (xi >> 1)
    y = lax.bitcast_convert_type(yi, jnp.float32)
    for _ in range(3):
        y = y * (1.5 - 0.5 * x * y * y)
    return y


def _sc_embed_ln(ids2, token_table, pos_table, ln_gamma, ln_beta):
    mesh = plsc.VectorSubcoreMesh(core_axis_name="c", subcore_axis_name="s")

    @functools.partial(
        pl.kernel,
        mesh=mesh,
        out_type=jax.ShapeDtypeStruct((BLOCKS, ROWS_PER_BLOCK, HIDDEN),
                                      jnp.float32),
        compiler_params=pltpu.CompilerParams(needs_layout_passes=False),
        scratch_types=[
            pltpu.VMEM((BLOCKS_PER_W, ROWS_PER_BLOCK), jnp.int32),
            pltpu.VMEM((NBUF, BUF_ROWS, HIDDEN), jnp.float32),
            pltpu.VMEM((NBUF, BUF_ROWS, HIDDEN), jnp.float32),
            pltpu.VMEM((POS_BUF, HIDDEN), jnp.float32),
            pltpu.VMEM((HIDDEN,), jnp.float32),
            pltpu.VMEM((HIDDEN,), jnp.float32),
            pltpu.VMEM((L * L,), jnp.float32),
            pltpu.VMEM((L * L,), jnp.float32),
            pltpu.VMEM((L,), jnp.float32),
            pltpu.VMEM((L,), jnp.float32),
            pltpu.SemaphoreType.DMA,
            pltpu.SemaphoreType.DMA,
            pltpu.SemaphoreType.DMA,
            pltpu.SemaphoreType.DMA,
        ],
    )
    def k(ids_hbm, table_hbm, pos_hbm, gamma_hbm, beta_hbm, out_hbm,
          idx_v, gbuf, obuf, pos_v, gamma_v, beta_v, smat_s, smat_q,
          stage_a, stage_b, gsem0, gsem1, ssem0, ssem1):
        wid = lax.axis_index("s") * 2 + lax.axis_index("c")
        gsems = (gsem0, gsem1)
        ssems = (ssem0, ssem1)

        pltpu.sync_copy(ids_hbm.at[pl.ds(wid * BLOCKS_PER_W, BLOCKS_PER_W)],
                        idx_v)
        pltpu.sync_copy(pos_hbm.at[pl.ds(0, SEQ)], pos_v.at[pl.ds(0, SEQ)])
        pltpu.sync_copy(gamma_hbm, gamma_v)
        pltpu.sync_copy(beta_hbm, beta_v)

        def gather_start(b, u):
            pltpu.make_async_copy(
                table_hbm.at[idx_v.at[u]],
                gbuf.at[b, pl.ds(0, ROWS_PER_BLOCK)],
                gsems[b]).start()

        def gather_wait(b):
            pltpu.make_async_copy(
                out_hbm.at[0],
                gbuf.at[b, pl.ds(0, ROWS_PER_BLOCK)],
                gsems[b]).wait()

        def scatter_start(b, blk):
            pltpu.make_async_copy(obuf.at[b, pl.ds(0, ROWS_PER_BLOCK)],
                                  out_hbm.at[blk], ssems[b]).start()

        def scatter_wait(b, blk):
            pltpu.make_async_copy(obuf.at[b, pl.ds(0, ROWS_PER_BLOCK)],
                                  out_hbm.at[blk], ssems[b]).wait()

        def compute(b, p0):
            def group_body(g, _):
                r0 = g * L
                for r in range(L):
                    row = r0 + r
                    acc_s = None
                    acc_q = None
                    for kk in range(KV):
                        sl = pl.ds(kk * L, L)
                        x = gbuf[b, row, sl] + pos_v[p0 + row, sl]
                        gbuf[b, row, sl] = x
                        acc_s = x if acc_s is None else acc_s + x
                        acc_q = x * x if acc_q is None else acc_q + x * x
                    smat_s[pl.ds(r * L, L)] = acc_s
                    smat_q[pl.ds(r * L, L)] = acc_q
                col_base = lax.iota(jnp.int32, L) * L
                sv = None
                qv = None
                for j in range(L):
                    cidx = col_base + j
                    cs = plsc.load_gather(smat_s, [cidx])
                    cq = plsc.load_gather(smat_q, [cidx])
                    sv = cs if sv is None else sv + cs
                    qv = cq if qv is None else qv + cq
                mean = sv * (1.0 / HIDDEN)
                var = qv * (1.0 / HIDDEN) - mean * mean
                rstd = _rsqrt(var + 1e-12)
                stage_a[...] = rstd
                stage_b[...] = -mean * rstd
                gv = [gamma_v[pl.ds(kk * L, L)] for kk in range(KV)]
                bv = [beta_v[pl.ds(kk * L, L)] for kk in range(KV)]
                av_all = stage_a[...]
                bv_all = stage_b[...]
                for r in range(L):
                    row = r0 + r
                    a = av_all[r]
                    bb = bv_all[r]
                    for kk in range(KV):
                        sl = pl.ds(kk * L, L)
                        x = gbuf[b, row, sl]
                        obuf[b, row, sl] = (x * a + bb) * gv[kk] + bv[kk]
                return 0

            lax.fori_loop(0, GROUPS, group_body, 0)

        for b in range(NBUF):
            gather_start(b, b)

        def super_body(su, _):
            for b in range(NBUF):
                u = su * NBUF + b
                blk = wid * BLOCKS_PER_W + u
                p0 = (u % 2) * ROWS_PER_BLOCK

                gather_wait(b)

                @pl.when(su >= 1)
                def _():
                    scatter_wait(b, blk - NBUF)

                compute(b, p0)
                scatter_start(b, blk)

                @pl.when(su <= SUPERS - 2)
                def _():
                    gather_start(b, u + NBUF)
            return 0

        lax.fori_loop(0, SUPERS, super_body, 0)

        for b in range(NBUF):
            u = (SUPERS - 1) * NBUF + b
            scatter_wait(b, wid * BLOCKS_PER_W + u)

    return k(ids2, token_table, pos_table, ln_gamma, ln_beta)


def kernel(input_ids, token_table, pos_table, ln_gamma, ln_beta):
    ids2 = input_ids.astype(jnp.int32).reshape(BLOCKS, ROWS_PER_BLOCK)
    out = _sc_embed_ln(ids2, token_table, pos_table, ln_gamma, ln_beta)
    return out.reshape(BATCH, SEQ, HIDDEN)
